# baseline (speedup 1.0000x reference)
import jax
import jax.numpy as jnp
from jax import lax
from jax.experimental import pallas as pl
from jax.experimental.pallas import tpu as pltpu

_STRIDES = (8.0, 16.0, 32.0)
_REG_MAX = 7
_TOP_K_N = 100
_IOU_THRESHOLD = 0.5
_BOX_SCORE = 0.3
_INP_H = 640.0
_INP_W = 640.0
_NMS_PRE = 1000

_C = 80
_N = 2432
_TP = 128
_NEG = -jnp.inf


def _nms_kernel(s_ref, x1_ref, y1_ref, x2_ref, y2_ref, o_ref,
                sw_ref, pm_ref, ss_ref, sb_ref):
    X1 = x1_ref[0]
    Y1 = y1_ref[0]
    X2 = x2_ref[0]
    Y2 = y2_ref[0]
    a2 = jnp.maximum(X2 - X1, 0.0) * jnp.maximum(Y2 - Y1, 0.0)

    sw_ref[...] = s_ref[0]
    ss_ref[...] = jnp.full((_C, _TP), _NEG, jnp.float32)

    iota_n = lax.broadcasted_iota(jnp.int32, (_C, _N), 1)
    col_tp = lax.broadcasted_iota(jnp.int32, (_C, _TP), 1)
    zpad4 = jnp.zeros((_C, 4), jnp.float32)

    def step(t, carry):
        s = sw_ref[...]
        bs = jnp.max(s, axis=1, keepdims=True)
        bi = jnp.min(jnp.where(s == bs, iota_n, _N), axis=1,
                     keepdims=True)
        oh = iota_n == bi
        bx1 = jnp.max(jnp.where(oh, X1, -1.0), axis=1, keepdims=True)
        by1 = jnp.max(jnp.where(oh, Y1, -1.0), axis=1, keepdims=True)
        bx2 = jnp.max(jnp.where(oh, X2, -1.0), axis=1, keepdims=True)
        by2 = jnp.max(jnp.where(oh, Y2, -1.0), axis=1, keepdims=True)
        ix1 = jnp.maximum(bx1, X1)
        iy1 = jnp.maximum(by1, Y1)
        ix2 = jnp.minimum(bx2, X2)
        iy2 = jnp.minimum(by2, Y2)
        inter = jnp.maximum(ix2 - ix1, 0.0) * jnp.maximum(iy2 - iy1, 0.0)
        a1 = jnp.maximum(bx2 - bx1, 0.0) * jnp.maximum(by2 - by1, 0.0)
        iou = inter / (a1 + a2 - inter + 1e-9)
        sw_ref[...] = jnp.where(iou > _IOU_THRESHOLD, _NEG, s)

        colm = col_tp == t
        ss_ref[...] = jnp.where(colm, bs, ss_ref[...])
        sb_ref[:, pl.ds(t, 1), :] = jnp.concatenate(
            [bx1, by1, bx2, by2, zpad4], axis=1)[:, None, :]
        return carry

    lax.fori_loop(0, _TOP_K_N, step, 0)

    selS = ss_ref[...]
    selS = jnp.where(jnp.isfinite(selS), selS, 0.0)
    pm_ref[...] = jnp.where(col_tp < _TOP_K_N, selS, _NEG)
    row_c1 = lax.broadcasted_iota(jnp.int32, (_C, 1), 0)
    row_ctp = lax.broadcasted_iota(jnp.int32, (_C, _TP), 0)
    lane8 = lax.broadcasted_iota(jnp.int32, (1, 8), 1)

    P = pm_ref[...]
    hv0 = P[:, 0:1]
    pt0 = jnp.zeros((_C, 1), jnp.int32)

    def mstep(t, carry):
        hv, ptrs = carry
        g = jnp.max(hv)
        rwin = jnp.min(jnp.where(hv == g, row_c1, _C))
        cwin = jnp.min(jnp.where(row_c1 == rwin, ptrs, _TP))
        q = sb_ref[pl.ds(rwin, 1), pl.ds(cwin, 1), :][0]
        q = jnp.where(lane8 == 4, g, q)
        q = jnp.where(lane8 == 5, rwin.astype(jnp.float32), q)
        q = jnp.where(g > _BOX_SCORE, q, 0.0)
        o_ref[0, pl.ds(t, 1), :] = q
        nextv = jnp.max(jnp.where((row_ctp == rwin) & (col_tp == cwin + 1),
                                  P, _NEG))
        hv = jnp.where(row_c1 == rwin, nextv, hv)
        ptrs = jnp.where(row_c1 == rwin, cwin + 1, ptrs)
        return hv, ptrs

    lax.fori_loop(0, _TOP_K_N, mstep, (hv0, pt0))


def _decode_level(cls_score, bbox_pred, stride):
    h, w, c = cls_score.shape
    scores = jax.nn.sigmoid(cls_score.reshape(-1, c))
    x = jax.nn.softmax(bbox_pred.reshape(-1, _REG_MAX + 1), axis=-1)
    ln = jnp.arange(_REG_MAX + 1, dtype=jnp.float32)[:, None]
    dist = (x @ ln).reshape(-1, 4) * stride
    y_range = (jnp.arange(h, dtype=jnp.float32) + 0.5) * stride
    x_range = (jnp.arange(w, dtype=jnp.float32) + 0.5) * stride
    yy = jnp.repeat(y_range, w)
    xx = jnp.tile(x_range, h)
    points = jnp.stack([yy, xx], axis=-1)
    if h * w > _NMS_PRE:
        max_scores = jnp.max(scores, axis=-1)
        _, topk = jax.lax.top_k(max_scores, _NMS_PRE)
        points = points[topk]
        dist = dist[topk]
        scores = scores[topk]
    y1 = jnp.clip(points[:, 0] - dist[:, 0], 0.0, _INP_H)
    x1 = jnp.clip(points[:, 1] - dist[:, 1], 0.0, _INP_W)
    y2 = jnp.clip(points[:, 0] + dist[:, 2], 0.0, _INP_H)
    x2 = jnp.clip(points[:, 1] + dist[:, 3], 0.0, _INP_W)
    return jnp.stack([x1, y1, x2, y2], axis=-1), scores


def _postprocess(cls0, box0, cls1, box1, cls2, box2, origin_shapes):
    B = cls0.shape[0]
    levels = [(cls0, box0, _STRIDES[0]), (cls1, box1, _STRIDES[1]),
              (cls2, box2, _STRIDES[2])]
    bb_list, sc_list = [], []
    for cls_l, box_l, s in levels:
        bb, sc = jax.vmap(lambda c, b, s=s: _decode_level(c, b, s))(cls_l, box_l)
        bb_list.append(bb)
        sc_list.append(sc)
    boxes = jnp.concatenate(bb_list, axis=1)
    scores = jnp.concatenate(sc_list, axis=1)
    n = boxes.shape[1]

    s_in = jnp.full((B, _C, _N), _NEG, jnp.float32)
    s_in = s_in.at[:, :, :n].set(scores.transpose(0, 2, 1))
    coords = []
    for k in range(4):
        c = jnp.zeros((B, 1, _N), jnp.float32)
        coords.append(c.at[:, 0, :n].set(boxes[..., k]))
    x1c, y1c, x2c, y2c = coords

    out = pl.pallas_call(
        _nms_kernel,
        grid=(B,),
        in_specs=[
            pl.BlockSpec((1, _C, _N), lambda b: (b, 0, 0)),
            pl.BlockSpec((1, 1, _N), lambda b: (b, 0, 0)),
            pl.BlockSpec((1, 1, _N), lambda b: (b, 0, 0)),
            pl.BlockSpec((1, 1, _N), lambda b: (b, 0, 0)),
            pl.BlockSpec((1, 1, _N), lambda b: (b, 0, 0)),
        ],
        out_specs=pl.BlockSpec((1, _TP, 8), lambda b: (b, 0, 0)),
        out_shape=jax.ShapeDtypeStruct((B, _TP, 8), jnp.float32),
        scratch_shapes=[
            pltpu.VMEM((_C, _N), jnp.float32),
            pltpu.VMEM((_C, _TP), jnp.float32),
            pltpu.VMEM((_C, _TP), jnp.float32),
            pltpu.VMEM((_C, _TP, 8), jnp.float32),
        ],
    )(s_in, x1c, y1c, x2c, y2c)
    return out[:, :_TOP_K_N, :6]


_postprocess_jit = jax.jit(_postprocess)


def kernel(cls0, box0, cls1, box1, cls2, box2, origin_shapes):
    return _postprocess_jit(cls0, box0, cls1, box1, cls2, box2, origin_shapes)


# speedup vs baseline: 1.2293x; 1.2293x over previous
import jax
import jax.numpy as jnp
from jax import lax
from jax.experimental import pallas as pl
from jax.experimental.pallas import tpu as pltpu

_STRIDES = (8.0, 16.0, 32.0)
_REG_MAX = 7
_TOP_K_N = 100
_IOU_THRESHOLD = 0.5
_BOX_SCORE = 0.3
_INP_H = 640.0
_INP_W = 640.0
_NMS_PRE = 1000

_C = 80
_P2 = 2
_C2 = _C * _P2
_N = 2432
_TP = 128
_NEG = -jnp.inf


def _nms_kernel(s_ref, x1_ref, y1_ref, x2_ref, y2_ref, o_ref,
                sw_ref, pm_ref, ss_ref, sb_ref,
                x1b_ref, y1b_ref, x2b_ref, y2b_ref, a2b_ref):
    def wide(cref):
        v = cref[0]
        return jnp.concatenate(
            [jnp.broadcast_to(v[0:1], (_C, _N)),
             jnp.broadcast_to(v[1:2], (_C, _N))], axis=0)

    x1b_ref[...] = wide(x1_ref)
    y1b_ref[...] = wide(y1_ref)
    x2b_ref[...] = wide(x2_ref)
    y2b_ref[...] = wide(y2_ref)
    X1 = x1b_ref[...]
    Y1 = y1b_ref[...]
    X2 = x2b_ref[...]
    Y2 = y2b_ref[...]
    a2b_ref[...] = (jnp.maximum(X2 - X1, 0.0) *
                    jnp.maximum(Y2 - Y1, 0.0))
    a2 = a2b_ref[...]

    sw_ref[...] = s_ref[0]
    ss_ref[...] = jnp.full((_C2, _TP), _NEG, jnp.float32)

    iota_n = lax.broadcasted_iota(jnp.int32, (_C2, _N), 1)
    col_tp = lax.broadcasted_iota(jnp.int32, (_C2, _TP), 1)
    zpad4 = jnp.zeros((_C2, 4), jnp.float32)

    def step(t, carry):
        s = sw_ref[...]
        bs = jnp.max(s, axis=1, keepdims=True)
        bi = jnp.min(jnp.where(s == bs, iota_n, _N), axis=1,
                     keepdims=True)
        oh = iota_n == bi
        bx1 = jnp.max(jnp.where(oh, X1, -1.0), axis=1, keepdims=True)
        by1 = jnp.max(jnp.where(oh, Y1, -1.0), axis=1, keepdims=True)
        bx2 = jnp.max(jnp.where(oh, X2, -1.0), axis=1, keepdims=True)
        by2 = jnp.max(jnp.where(oh, Y2, -1.0), axis=1, keepdims=True)
        ix1 = jnp.maximum(bx1, X1)
        iy1 = jnp.maximum(by1, Y1)
        ix2 = jnp.minimum(bx2, X2)
        iy2 = jnp.minimum(by2, Y2)
        inter = jnp.maximum(ix2 - ix1, 0.0) * jnp.maximum(iy2 - iy1, 0.0)
        a1 = jnp.maximum(bx2 - bx1, 0.0) * jnp.maximum(by2 - by1, 0.0)
        iou = inter / (a1 + a2 - inter + 1e-9)
        sw_ref[...] = jnp.where(iou > _IOU_THRESHOLD, _NEG, s)

        colm = col_tp == t
        ss_ref[...] = jnp.where(colm, bs, ss_ref[...])
        sb_ref[:, pl.ds(t, 1), :] = jnp.concatenate(
            [bx1, by1, bx2, by2, zpad4], axis=1)[:, None, :]
        return carry

    lax.fori_loop(0, _TOP_K_N, step, 0)

    selS = ss_ref[...]
    selS = jnp.where(jnp.isfinite(selS), selS, 0.0)
    pm_ref[...] = jnp.where(col_tp < _TOP_K_N, selS, _NEG)
    row_c1 = lax.broadcasted_iota(jnp.int32, (_C, 1), 0)
    row_ctp = lax.broadcasted_iota(jnp.int32, (_C, _TP), 0)
    col_ctp = lax.broadcasted_iota(jnp.int32, (_C, _TP), 1)
    lane8 = lax.broadcasted_iota(jnp.int32, (1, 8), 1)

    def half(p, base, rwoff):
        rm = jnp.max(p, axis=1, keepdims=True)
        coli = jnp.min(jnp.where(p == rm, col_ctp, _TP), axis=1,
                       keepdims=True)
        g = jnp.max(rm)
        rwin = jnp.min(jnp.where(rm == g, row_c1, _C))
        cwin = jnp.min(jnp.where(row_c1 == rwin, coli, _TP))
        pnew = jnp.where((row_ctp == rwin) & (col_ctp == cwin), _NEG, p)
        q = sb_ref[pl.ds(rwin + rwoff, 1), pl.ds(cwin, 1), :][0]
        q = jnp.where(lane8 == 4, g, q)
        q = jnp.where(lane8 == 5, rwin.astype(jnp.float32), q)
        q = jnp.where(g > _BOX_SCORE, q, 0.0)
        return pnew, q

    def mstep(t, carry):
        pa, pb = carry
        pa, qa = half(pa, 0, 0)
        pb, qb = half(pb, 0, _C)
        o_ref[0, 0, pl.ds(t, 1), :] = qa
        o_ref[0, 1, pl.ds(t, 1), :] = qb
        return pa, pb

    P = pm_ref[...]
    lax.fori_loop(0, _TOP_K_N, mstep, (P[:_C], P[_C:]))


def _decode_level(cls_score, bbox_pred, stride):
    h, w, c = cls_score.shape
    scores = jax.nn.sigmoid(cls_score.reshape(-1, c))
    x = jax.nn.softmax(bbox_pred.reshape(-1, _REG_MAX + 1), axis=-1)
    ln = jnp.arange(_REG_MAX + 1, dtype=jnp.float32)[:, None]
    dist = (x @ ln).reshape(-1, 4) * stride
    y_range = (jnp.arange(h, dtype=jnp.float32) + 0.5) * stride
    x_range = (jnp.arange(w, dtype=jnp.float32) + 0.5) * stride
    yy = jnp.repeat(y_range, w)
    xx = jnp.tile(x_range, h)
    points = jnp.stack([yy, xx], axis=-1)
    if h * w > _NMS_PRE:
        max_scores = jnp.max(scores, axis=-1)
        _, topk = jax.lax.top_k(max_scores, _NMS_PRE)
        points = points[topk]
        dist = dist[topk]
        scores = scores[topk]
    y1 = jnp.clip(points[:, 0] - dist[:, 0], 0.0, _INP_H)
    x1 = jnp.clip(points[:, 1] - dist[:, 1], 0.0, _INP_W)
    y2 = jnp.clip(points[:, 0] + dist[:, 2], 0.0, _INP_H)
    x2 = jnp.clip(points[:, 1] + dist[:, 3], 0.0, _INP_W)
    return jnp.stack([x1, y1, x2, y2], axis=-1), scores


def _postprocess(cls0, box0, cls1, box1, cls2, box2, origin_shapes):
    B = cls0.shape[0]
    G = B // _P2
    levels = [(cls0, box0, _STRIDES[0]), (cls1, box1, _STRIDES[1]),
              (cls2, box2, _STRIDES[2])]
    bb_list, sc_list = [], []
    for cls_l, box_l, s in levels:
        bb, sc = jax.vmap(lambda c, b, s=s: _decode_level(c, b, s))(cls_l, box_l)
        bb_list.append(bb)
        sc_list.append(sc)
    boxes = jnp.concatenate(bb_list, axis=1)
    scores = jnp.concatenate(sc_list, axis=1)
    n = boxes.shape[1]

    s_in = jnp.full((B, _C, _N), _NEG, jnp.float32)
    s_in = s_in.at[:, :, :n].set(scores.transpose(0, 2, 1))
    s_in = s_in.reshape(G, _C2, _N)
    coords = []
    for k in range(4):
        c = jnp.zeros((B, 1, _N), jnp.float32)
        coords.append(c.at[:, 0, :n].set(boxes[..., k]).reshape(G, _P2, _N))
    x1c, y1c, x2c, y2c = coords

    out = pl.pallas_call(
        _nms_kernel,
        grid=(G,),
        in_specs=[
            pl.BlockSpec((1, _C2, _N), lambda b: (b, 0, 0)),
            pl.BlockSpec((1, _P2, _N), lambda b: (b, 0, 0)),
            pl.BlockSpec((1, _P2, _N), lambda b: (b, 0, 0)),
            pl.BlockSpec((1, _P2, _N), lambda b: (b, 0, 0)),
            pl.BlockSpec((1, _P2, _N), lambda b: (b, 0, 0)),
        ],
        out_specs=pl.BlockSpec((1, _P2, _TP, 8), lambda b: (b, 0, 0, 0)),
        out_shape=jax.ShapeDtypeStruct((G, _P2, _TP, 8), jnp.float32),
        scratch_shapes=[
            pltpu.VMEM((_C2, _N), jnp.float32),
            pltpu.VMEM((_C2, _TP), jnp.float32),
            pltpu.VMEM((_C2, _TP), jnp.float32),
            pltpu.VMEM((_C2, _TP, 8), jnp.float32),
            pltpu.VMEM((_C2, _N), jnp.float32),
            pltpu.VMEM((_C2, _N), jnp.float32),
            pltpu.VMEM((_C2, _N), jnp.float32),
            pltpu.VMEM((_C2, _N), jnp.float32),
            pltpu.VMEM((_C2, _N), jnp.float32),
        ],
    )(s_in, x1c, y1c, x2c, y2c)
    return out.reshape(B, _TP, 8)[:, :_TOP_K_N, :6]


_postprocess_jit = jax.jit(_postprocess)


def kernel(cls0, box0, cls1, box1, cls2, box2, origin_shapes):
    return _postprocess_jit(cls0, box0, cls1, box1, cls2, box2, origin_shapes)


# speedup vs baseline: 1.3093x; 1.0651x over previous
import jax
import jax.numpy as jnp
from jax import lax
from jax.experimental import pallas as pl
from jax.experimental.pallas import tpu as pltpu

_STRIDES = (8.0, 16.0, 32.0)
_REG_MAX = 7
_TOP_K_N = 100
_IOU_THRESHOLD = 0.5
_BOX_SCORE = 0.3
_INP_H = 640.0
_INP_W = 640.0
_NMS_PRE = 1000

_C = 80
_P2 = 4
_C2 = _C * _P2
_N = 2432
_TP = 128
_NEG = -jnp.inf


def _nms_kernel(s_ref, x1_ref, y1_ref, x2_ref, y2_ref, o_ref,
                sw_ref, pm_ref, ss_ref, sb_ref,
                x1b_ref, y1b_ref, x2b_ref, y2b_ref, a2b_ref):
    def wide(cref):
        v = cref[0]
        return jnp.concatenate(
            [jnp.broadcast_to(v[i:i + 1], (_C, _N))
             for i in range(_P2)], axis=0)

    x1b_ref[...] = wide(x1_ref)
    y1b_ref[...] = wide(y1_ref)
    x2b_ref[...] = wide(x2_ref)
    y2b_ref[...] = wide(y2_ref)
    X1 = x1b_ref[...]
    Y1 = y1b_ref[...]
    X2 = x2b_ref[...]
    Y2 = y2b_ref[...]
    a2b_ref[...] = (jnp.maximum(X2 - X1, 0.0) *
                    jnp.maximum(Y2 - Y1, 0.0))
    a2 = a2b_ref[...]

    sw_ref[...] = s_ref[0]
    ss_ref[...] = jnp.full((_C2, _TP), _NEG, jnp.float32)

    iota_n = lax.broadcasted_iota(jnp.int32, (_C2, _N), 1)
    col_tp = lax.broadcasted_iota(jnp.int32, (_C2, _TP), 1)
    zpad4 = jnp.zeros((_C2, 4), jnp.float32)

    def step(t, carry):
        s = sw_ref[...]
        bs = jnp.max(s, axis=1, keepdims=True)
        bi = jnp.min(jnp.where(s == bs, iota_n, _N), axis=1,
                     keepdims=True)
        oh = iota_n == bi
        bx1 = jnp.max(jnp.where(oh, X1, -1.0), axis=1, keepdims=True)
        by1 = jnp.max(jnp.where(oh, Y1, -1.0), axis=1, keepdims=True)
        bx2 = jnp.max(jnp.where(oh, X2, -1.0), axis=1, keepdims=True)
        by2 = jnp.max(jnp.where(oh, Y2, -1.0), axis=1, keepdims=True)
        ix1 = jnp.maximum(bx1, X1)
        iy1 = jnp.maximum(by1, Y1)
        ix2 = jnp.minimum(bx2, X2)
        iy2 = jnp.minimum(by2, Y2)
        inter = jnp.maximum(ix2 - ix1, 0.0) * jnp.maximum(iy2 - iy1, 0.0)
        a1 = jnp.maximum(bx2 - bx1, 0.0) * jnp.maximum(by2 - by1, 0.0)
        iou = inter / (a1 + a2 - inter + 1e-9)
        sw_ref[...] = jnp.where(iou > _IOU_THRESHOLD, _NEG, s)

        colm = col_tp == t
        ss_ref[...] = jnp.where(colm, bs, ss_ref[...])
        sb_ref[:, pl.ds(t, 1), :] = jnp.concatenate(
            [bx1, by1, bx2, by2, zpad4], axis=1)[:, None, :]
        return carry

    lax.fori_loop(0, _TOP_K_N, step, 0)

    selS = ss_ref[...]
    selS = jnp.where(jnp.isfinite(selS), selS, 0.0)
    pm_ref[...] = jnp.where(col_tp < _TOP_K_N, selS, _NEG)
    row_c1 = lax.broadcasted_iota(jnp.int32, (_C, 1), 0)
    row_ctp = lax.broadcasted_iota(jnp.int32, (_C, _TP), 0)
    col_ctp = lax.broadcasted_iota(jnp.int32, (_C, _TP), 1)
    lane8 = lax.broadcasted_iota(jnp.int32, (1, 8), 1)

    def half(p, base, rwoff):
        rm = jnp.max(p, axis=1, keepdims=True)
        coli = jnp.min(jnp.where(p == rm, col_ctp, _TP), axis=1,
                       keepdims=True)
        g = jnp.max(rm)
        rwin = jnp.min(jnp.where(rm == g, row_c1, _C))
        cwin = jnp.min(jnp.where(row_c1 == rwin, coli, _TP))
        pnew = jnp.where((row_ctp == rwin) & (col_ctp == cwin), _NEG, p)
        q = sb_ref[pl.ds(rwin + rwoff, 1), pl.ds(cwin, 1), :][0]
        q = jnp.where(lane8 == 4, g, q)
        q = jnp.where(lane8 == 5, rwin.astype(jnp.float32), q)
        q = jnp.where(g > _BOX_SCORE, q, 0.0)
        return pnew, q

    def mstep(t, carry):
        newps = []
        for i in range(_P2):
            pi, qi = half(carry[i], 0, i * _C)
            o_ref[0, i, pl.ds(t, 1), :] = qi
            newps.append(pi)
        return tuple(newps)

    P = pm_ref[...]
    lax.fori_loop(0, _TOP_K_N, mstep,
                  tuple(P[i * _C:(i + 1) * _C] for i in range(_P2)))


def _decode_level(cls_score, bbox_pred, stride):
    h, w, c = cls_score.shape
    scores = jax.nn.sigmoid(cls_score.reshape(-1, c))
    x = jax.nn.softmax(bbox_pred.reshape(-1, _REG_MAX + 1), axis=-1)
    ln = jnp.arange(_REG_MAX + 1, dtype=jnp.float32)[:, None]
    dist = (x @ ln).reshape(-1, 4) * stride
    y_range = (jnp.arange(h, dtype=jnp.float32) + 0.5) * stride
    x_range = (jnp.arange(w, dtype=jnp.float32) + 0.5) * stride
    yy = jnp.repeat(y_range, w)
    xx = jnp.tile(x_range, h)
    points = jnp.stack([yy, xx], axis=-1)
    if h * w > _NMS_PRE:
        max_scores = jnp.max(scores, axis=-1)
        _, topk = jax.lax.top_k(max_scores, _NMS_PRE)
        points = points[topk]
        dist = dist[topk]
        scores = scores[topk]
    y1 = jnp.clip(points[:, 0] - dist[:, 0], 0.0, _INP_H)
    x1 = jnp.clip(points[:, 1] - dist[:, 1], 0.0, _INP_W)
    y2 = jnp.clip(points[:, 0] + dist[:, 2], 0.0, _INP_H)
    x2 = jnp.clip(points[:, 1] + dist[:, 3], 0.0, _INP_W)
    return jnp.stack([x1, y1, x2, y2], axis=-1), scores


def _postprocess(cls0, box0, cls1, box1, cls2, box2, origin_shapes):
    B = cls0.shape[0]
    G = B // _P2
    levels = [(cls0, box0, _STRIDES[0]), (cls1, box1, _STRIDES[1]),
              (cls2, box2, _STRIDES[2])]
    bb_list, sc_list = [], []
    for cls_l, box_l, s in levels:
        bb, sc = jax.vmap(lambda c, b, s=s: _decode_level(c, b, s))(cls_l, box_l)
        bb_list.append(bb)
        sc_list.append(sc)
    boxes = jnp.concatenate(bb_list, axis=1)
    scores = jnp.concatenate(sc_list, axis=1)
    n = boxes.shape[1]

    s_in = jnp.full((B, _C, _N), _NEG, jnp.float32)
    s_in = s_in.at[:, :, :n].set(scores.transpose(0, 2, 1))
    s_in = s_in.reshape(G, _C2, _N)
    coords = []
    for k in range(4):
        c = jnp.zeros((B, 1, _N), jnp.float32)
        coords.append(c.at[:, 0, :n].set(boxes[..., k]).reshape(G, _P2, _N))
    x1c, y1c, x2c, y2c = coords

    out = pl.pallas_call(
        _nms_kernel,
        grid=(G,),
        in_specs=[
            pl.BlockSpec((1, _C2, _N), lambda b: (b, 0, 0)),
            pl.BlockSpec((1, _P2, _N), lambda b: (b, 0, 0)),
            pl.BlockSpec((1, _P2, _N), lambda b: (b, 0, 0)),
            pl.BlockSpec((1, _P2, _N), lambda b: (b, 0, 0)),
            pl.BlockSpec((1, _P2, _N), lambda b: (b, 0, 0)),
        ],
        out_specs=pl.BlockSpec((1, _P2, _TP, 8), lambda b: (b, 0, 0, 0)),
        out_shape=jax.ShapeDtypeStruct((G, _P2, _TP, 8), jnp.float32),
        scratch_shapes=[
            pltpu.VMEM((_C2, _N), jnp.float32),
            pltpu.VMEM((_C2, _TP), jnp.float32),
            pltpu.VMEM((_C2, _TP), jnp.float32),
            pltpu.VMEM((_C2, _TP, 8), jnp.float32),
            pltpu.VMEM((_C2, _N), jnp.float32),
            pltpu.VMEM((_C2, _N), jnp.float32),
            pltpu.VMEM((_C2, _N), jnp.float32),
            pltpu.VMEM((_C2, _N), jnp.float32),
            pltpu.VMEM((_C2, _N), jnp.float32),
        ],
    )(s_in, x1c, y1c, x2c, y2c)
    return out.reshape(B, _TP, 8)[:, :_TOP_K_N, :6]


_postprocess_jit = jax.jit(_postprocess)


def kernel(cls0, box0, cls1, box1, cls2, box2, origin_shapes):
    return _postprocess_jit(cls0, box0, cls1, box1, cls2, box2, origin_shapes)
